# Initial kernel scaffold; baseline (speedup 1.0000x reference)
import functools

import jax
import jax.numpy as jnp
from jax import lax
from jax.experimental import pallas as pl
from jax.experimental.pallas import tpu as pltpu
from jax.experimental.pallas import tpu_sc as plsc

N = 10000
E = 320000
R = 8
D = 128
H = 128
C = 16

NB = 400
NBLK = N // NB

NW = 32
EPW = E // NW
K = 80
NCH = EPW // K
TROWS = N // 16


def _mm9_body(x_ref, w_ref, out_ref):
    x = x_ref[...]
    for r in range(9):
        out_ref[r] = jnp.dot(x, w_ref[r], preferred_element_type=jnp.float32)


def _transform1(x, w1cat):
    return pl.pallas_call(
        _mm9_body,
        grid=(NBLK,),
        in_specs=[
            pl.BlockSpec((NB, D), lambda i: (i, 0)),
            pl.BlockSpec((9, D, H), lambda i: (0, 0, 0)),
        ],
        out_specs=pl.BlockSpec((9, NB, H), lambda i: (0, i, 0)),
        out_shape=jax.ShapeDtypeStruct((9, N, H), jnp.float32),
    )(x, w1cat)


def _combine_body(p_ref, cnt_ref, xr_ref, b1_ref, w2_ref, out_ref):
    cnt = cnt_ref[0, :, 0:1] + cnt_ref[1, :, 0:1]
    h = (p_ref[0] + p_ref[1]) / jnp.maximum(cnt, 1.0)
    h = h + xr_ref[...] + b1_ref[...]
    h = jnp.maximum(h, 0.0)
    for r in range(9):
        out_ref[r] = jnp.dot(h, w2_ref[r], preferred_element_type=jnp.float32)


def _combine(p1, cnt, xroot, b1, w2cat):
    return pl.pallas_call(
        _combine_body,
        grid=(NBLK,),
        in_specs=[
            pl.BlockSpec((2, NB, H), lambda i: (0, i, 0)),
            pl.BlockSpec((2, NB, C), lambda i: (0, i, 0)),
            pl.BlockSpec((NB, H), lambda i: (i, 0)),
            pl.BlockSpec((1, H), lambda i: (0, 0)),
            pl.BlockSpec((9, H, C), lambda i: (0, 0, 0)),
        ],
        out_specs=pl.BlockSpec((9, NB, C), lambda i: (0, i, 0)),
        out_shape=jax.ShapeDtypeStruct((9, N, C), jnp.float32),
    )(p1, cnt, xroot, b1, w2cat)


def _final_body(p_ref, cnt_ref, hr_ref, b2_ref, out_ref):
    cnt = cnt_ref[0, :, 0:1] + cnt_ref[1, :, 0:1]
    z = (p_ref[0] + p_ref[1]) / jnp.maximum(cnt, 1.0)
    z = z + hr_ref[...] + b2_ref[...]
    m = jnp.max(z, axis=1, keepdims=True)
    e = jnp.exp(z - m)
    lse = jnp.log(jnp.sum(e, axis=1, keepdims=True))
    out_ref[...] = z - m - lse


def _final(p2, cnt, h2root, b2):
    return pl.pallas_call(
        _final_body,
        grid=(NBLK,),
        in_specs=[
            pl.BlockSpec((2, NB, C), lambda i: (0, i, 0)),
            pl.BlockSpec((2, NB, C), lambda i: (0, i, 0)),
            pl.BlockSpec((NB, C), lambda i: (i, 0)),
            pl.BlockSpec((1, C), lambda i: (0, 0)),
        ],
        out_specs=pl.BlockSpec((NB, C), lambda i: (i, 0)),
        out_shape=jax.ShapeDtypeStruct((N, C), jnp.float32),
    )(p2, cnt, h2root, b2)


_MESH = plsc.VectorSubcoreMesh(core_axis_name="c", subcore_axis_name="s")


def _agg1_body(table, fidx, dsts, zh, z16,
               out_p, out_cnt,
               idx_v, dst_v, rows_v, ones_v, acc_sh, cnt_sh, sem):
    c = lax.axis_index("c")
    s = lax.axis_index("s")
    wid = c * 16 + s

    pltpu.sync_copy(zh.at[pl.ds(s * TROWS, TROWS)],
                    acc_sh.at[pl.ds(s * TROWS, TROWS)])
    pltpu.sync_copy(z16.at[pl.ds(s * TROWS, TROWS)],
                    cnt_sh.at[pl.ds(s * TROWS, TROWS)])
    for j in range(K):
        ones_v[j] = jnp.ones((16,), jnp.float32)
    plsc.subcore_barrier()

    def body(ci, _):
        base = wid * EPW + ci * K
        pltpu.sync_copy(fidx.at[pl.ds(base, K)], idx_v)
        pltpu.sync_copy(dsts.at[pl.ds(base, K)], dst_v)
        pltpu.async_copy(table.at[idx_v], rows_v, sem).wait()
        pltpu.sync_copy(rows_v, acc_sh.at[dst_v], add=True)
        pltpu.sync_copy(ones_v, cnt_sh.at[dst_v], add=True)
        return 0

    lax.fori_loop(0, NCH, body, 0)
    plsc.subcore_barrier()

    pltpu.sync_copy(acc_sh.at[pl.ds(s * TROWS, TROWS)],
                    out_p.at[c, pl.ds(s * TROWS, TROWS)])
    pltpu.sync_copy(cnt_sh.at[pl.ds(s * TROWS, TROWS)],
                    out_cnt.at[c, pl.ds(s * TROWS, TROWS)])


_agg1 = pl.kernel(
    _agg1_body,
    out_type=(jax.ShapeDtypeStruct((2, N, H), jnp.float32),
              jax.ShapeDtypeStruct((2, N, C), jnp.float32)),
    mesh=_MESH,
    scratch_types=[
        pltpu.VMEM((K,), jnp.int32),
        pltpu.VMEM((K,), jnp.int32),
        pltpu.VMEM((K, H), jnp.float32),
        pltpu.VMEM((K, C), jnp.float32),
        pltpu.VMEM_SHARED((N, H), jnp.float32),
        pltpu.VMEM_SHARED((N, C), jnp.float32),
        pltpu.SemaphoreType.DMA,
    ],
)


def _agg2_body(table, fidx, dsts, z16,
               out_p,
               idx_v, dst_v, rows_v, acc_sh, sem):
    c = lax.axis_index("c")
    s = lax.axis_index("s")
    wid = c * 16 + s

    pltpu.sync_copy(z16.at[pl.ds(s * TROWS, TROWS)],
                    acc_sh.at[pl.ds(s * TROWS, TROWS)])
    plsc.subcore_barrier()

    def body(ci, _):
        base = wid * EPW + ci * K
        pltpu.sync_copy(fidx.at[pl.ds(base, K)], idx_v)
        pltpu.sync_copy(dsts.at[pl.ds(base, K)], dst_v)
        pltpu.async_copy(table.at[idx_v], rows_v, sem).wait()
        pltpu.sync_copy(rows_v, acc_sh.at[dst_v], add=True)
        return 0

    lax.fori_loop(0, NCH, body, 0)
    plsc.subcore_barrier()

    pltpu.sync_copy(acc_sh.at[pl.ds(s * TROWS, TROWS)],
                    out_p.at[c, pl.ds(s * TROWS, TROWS)])


_agg2 = pl.kernel(
    _agg2_body,
    out_type=jax.ShapeDtypeStruct((2, N, C), jnp.float32),
    mesh=_MESH,
    scratch_types=[
        pltpu.VMEM((K,), jnp.int32),
        pltpu.VMEM((K,), jnp.int32),
        pltpu.VMEM((K, C), jnp.float32),
        pltpu.VMEM_SHARED((N, C), jnp.float32),
        pltpu.SemaphoreType.DMA,
    ],
)


@jax.jit
def kernel(x, edge_index, edge_type, W1, root1, b1, W2, root2, b2):
    src = edge_index[0]
    dst = edge_index[1]
    fidx = edge_type * N + src

    w1cat = jnp.concatenate([W1, root1[None]], axis=0)
    w2cat = jnp.concatenate([W2, root2[None]], axis=0)

    zh = jnp.zeros((N, H), jnp.float32)
    z16 = jnp.zeros((N, C), jnp.float32)

    h1t = _transform1(x, w1cat)
    p1, cnt = _agg1(h1t.reshape(9 * N, H), fidx, dst, zh, z16)
    h2t = _combine(p1, cnt, h1t[8], b1.reshape(1, H), w2cat)
    p2 = _agg2(h2t.reshape(9 * N, C), fidx, dst, z16)
    return _final(p2, cnt, h2t[8], b2.reshape(1, C))


# speedup vs baseline: 13.4675x; 13.4675x over previous
import functools

import jax
import jax.numpy as jnp
from jax import lax
from jax.experimental import pallas as pl
from jax.experimental.pallas import tpu as pltpu
from jax.experimental.pallas import tpu_sc as plsc

N = 10000
E = 320000
R = 8
D = 128
H = 128
C = 16

NB = 400
NBLK = N // NB

NW = 32
EPW = E // NW
K = 80
NCH = EPW // K
NPAD = 10240
TROWS = NPAD // 16


def _mm9_body(x_ref, w_ref, out_ref):
    x = x_ref[...]
    for r in range(9):
        out_ref[r] = jnp.dot(x, w_ref[r], preferred_element_type=jnp.float32)


def _transform1(x, w1cat):
    return pl.pallas_call(
        _mm9_body,
        grid=(NBLK,),
        in_specs=[
            pl.BlockSpec((NB, D), lambda i: (i, 0)),
            pl.BlockSpec((9, D, H), lambda i: (0, 0, 0)),
        ],
        out_specs=pl.BlockSpec((9, NB, H), lambda i: (0, i, 0)),
        out_shape=jax.ShapeDtypeStruct((9, N, H), jnp.float32),
    )(x, w1cat)


def _combine_body(p_ref, cnt_ref, xr_ref, b1_ref, w2p_ref, r2_ref,
                  big_ref, hr_ref):
    cnt = cnt_ref[0, :, 0:1] + cnt_ref[1, :, 0:1]
    h = (p_ref[0] + p_ref[1]) / jnp.maximum(cnt, 1.0)
    h = h + xr_ref[...] + b1_ref[...]
    h = jnp.maximum(h, 0.0)
    for r in range(R):
        big_ref[r] = jnp.dot(h, w2p_ref[r], preferred_element_type=jnp.float32)
    hr_ref[...] = jnp.dot(h, r2_ref[...], preferred_element_type=jnp.float32)


def _combine(p1, cnt, xroot, b1, w2pad, root2):
    return pl.pallas_call(
        _combine_body,
        grid=(NBLK,),
        in_specs=[
            pl.BlockSpec((2, NB, H), lambda i: (0, i, 0)),
            pl.BlockSpec((2, NB, 8), lambda i: (0, i, 0)),
            pl.BlockSpec((NB, H), lambda i: (i, 0)),
            pl.BlockSpec((1, H), lambda i: (0, 0)),
            pl.BlockSpec((R, H, H), lambda i: (0, 0, 0)),
            pl.BlockSpec((H, C), lambda i: (0, 0)),
        ],
        out_specs=(pl.BlockSpec((R, NB, H), lambda i: (0, i, 0)),
                   pl.BlockSpec((NB, C), lambda i: (i, 0))),
        out_shape=(jax.ShapeDtypeStruct((R, N, H), jnp.float32),
                   jax.ShapeDtypeStruct((N, C), jnp.float32)),
    )(p1, cnt, xroot, b1, w2pad, root2)


def _final_body(p_ref, cnt_ref, hr_ref, b2_ref, out_ref):
    cnt = cnt_ref[0, :, 0:1] + cnt_ref[1, :, 0:1]
    s = p_ref[0] + p_ref[1]
    m16 = s[:, 0:C]
    for g in range(1, R):
        m16 = m16 + s[:, g * C:(g + 1) * C]
    z = m16 / jnp.maximum(cnt, 1.0)
    z = z + hr_ref[...] + b2_ref[...]
    m = jnp.max(z, axis=1, keepdims=True)
    e = jnp.exp(z - m)
    lse = jnp.log(jnp.sum(e, axis=1, keepdims=True))
    out_ref[...] = z - m - lse


def _final(p2, cnt, h2root, b2):
    return pl.pallas_call(
        _final_body,
        grid=(NBLK,),
        in_specs=[
            pl.BlockSpec((2, NB, H), lambda i: (0, i, 0)),
            pl.BlockSpec((2, NB, 8), lambda i: (0, i, 0)),
            pl.BlockSpec((NB, C), lambda i: (i, 0)),
            pl.BlockSpec((1, C), lambda i: (0, 0)),
        ],
        out_specs=pl.BlockSpec((NB, C), lambda i: (i, 0)),
        out_shape=jax.ShapeDtypeStruct((N, C), jnp.float32),
    )(p2, cnt, h2root, b2)


def _agg_body(table, fidx, dsts, zh, out_p,
              idx_v, dst_v, rows_v, acc_sh, sem):
    c = lax.axis_index("c")
    s = lax.axis_index("s")
    wid = c * 16 + s

    pltpu.sync_copy(zh.at[pl.ds(s * TROWS, TROWS)],
                    acc_sh.at[pl.ds(s * TROWS, TROWS)])
    plsc.subcore_barrier()

    def body(ci, _):
        base = wid * EPW + ci * K
        pltpu.sync_copy(fidx.at[pl.ds(base, K)], idx_v)
        pltpu.sync_copy(dsts.at[pl.ds(base, K)], dst_v)
        pltpu.async_copy(table.at[idx_v], rows_v, sem).wait()
        pltpu.sync_copy(rows_v, acc_sh.at[dst_v], add=True)
        return 0

    lax.fori_loop(0, NCH, body, 0)
    plsc.subcore_barrier()

    pltpu.sync_copy(acc_sh.at[pl.ds(s * TROWS, TROWS)],
                    out_p.at[pl.ds(c * NPAD + s * TROWS, TROWS)])


@functools.cache
def _get_agg():
    return pl.kernel(
        _agg_body,
        out_type=jax.ShapeDtypeStruct((2 * NPAD, H), jnp.float32),
        mesh=plsc.VectorSubcoreMesh(core_axis_name="c", subcore_axis_name="s"),
        scratch_types=[
            pltpu.VMEM((K,), jnp.int32),
            pltpu.VMEM((K,), jnp.int32),
            pltpu.VMEM((K, H), jnp.float32),
            pltpu.VMEM_SHARED((NPAD, H), jnp.float32),
            pltpu.SemaphoreType.DMA,
        ],
    )


def _cnt_body(dsts, zh, out_cnt, dst_v, ones_v, cnt_sh, sem):
    c = lax.axis_index("c")
    s = lax.axis_index("s")
    wid = c * 16 + s

    pltpu.sync_copy(zh.at[pl.ds(s * TROWS, TROWS)],
                    cnt_sh.at[pl.ds(s * TROWS, TROWS)])
    for j in range(K):
        for q in range(H // 16):
            ones_v[j, pl.ds(q * 16, 16)] = jnp.ones((16,), jnp.float32)
    plsc.subcore_barrier()

    def body(ci, _):
        base = wid * EPW + ci * K
        pltpu.sync_copy(dsts.at[pl.ds(base, K)], dst_v)
        pltpu.sync_copy(ones_v, cnt_sh.at[dst_v], add=True)
        return 0

    lax.fori_loop(0, NCH, body, 0)
    plsc.subcore_barrier()

    pltpu.sync_copy(cnt_sh.at[pl.ds(s * TROWS, TROWS)],
                    out_cnt.at[pl.ds(c * NPAD + s * TROWS, TROWS)])


@functools.cache
def _get_cnt():
    return pl.kernel(
        _cnt_body,
        out_type=jax.ShapeDtypeStruct((2 * NPAD, H), jnp.float32),
        mesh=plsc.VectorSubcoreMesh(core_axis_name="c", subcore_axis_name="s"),
        scratch_types=[
            pltpu.VMEM((K,), jnp.int32),
            pltpu.VMEM((K, H), jnp.float32),
            pltpu.VMEM_SHARED((NPAD, H), jnp.float32),
            pltpu.SemaphoreType.DMA,
        ],
    )


@jax.jit
def kernel(x, edge_index, edge_type, W1, root1, b1, W2, root2, b2):
    src = edge_index[0]
    dst = edge_index[1]
    fidx = edge_type * N + src

    w1cat = jnp.concatenate([W1, root1[None]], axis=0)
    eye = jnp.eye(R, dtype=jnp.float32)
    w2pad = jnp.einsum('rhc,rg->rhgc', W2, eye).reshape(R, H, R * C)

    zh = jnp.zeros((NPAD, H), jnp.float32)

    cntp = _get_cnt()(dst, zh)
    cnt = cntp.reshape(2, NPAD, H)[:, :, 0:8]

    h1t = _transform1(x, w1cat)
    p1 = _get_agg()(h1t.reshape(9 * N, H), fidx, dst, zh)
    big, h2root = _combine(p1.reshape(2, NPAD, H), cnt, h1t[8],
                           b1.reshape(1, H), w2pad, root2)
    p2 = _get_agg()(big.reshape(R * N, H), fidx, dst, zh)
    return _final(p2.reshape(2, NPAD, H), cnt, h2root, b2.reshape(1, C))


# speedup vs baseline: 15.2760x; 1.1343x over previous
import functools

import jax
import jax.numpy as jnp
from jax import lax
from jax.experimental import pallas as pl
from jax.experimental.pallas import tpu as pltpu
from jax.experimental.pallas import tpu_sc as plsc

N = 10000
E = 320000
R = 8
D = 128
H = 128
C = 16

NB = 400
NBLK = N // NB

NW = 32
EPW = E // NW
K = 80
NCH = EPW // K
NPAD = 10240
TROWS = NPAD // 16


def _mm9_body(x_ref, w_ref, out_ref):
    x = x_ref[...]
    for r in range(9):
        out_ref[r] = jnp.dot(x, w_ref[r], preferred_element_type=jnp.float32)


def _transform1(x, w1cat):
    return pl.pallas_call(
        _mm9_body,
        grid=(NBLK,),
        in_specs=[
            pl.BlockSpec((NB, D), lambda i: (i, 0)),
            pl.BlockSpec((9, D, H), lambda i: (0, 0, 0)),
        ],
        out_specs=pl.BlockSpec((9, NB, H), lambda i: (0, i, 0)),
        out_shape=jax.ShapeDtypeStruct((9, N, H), jnp.float32),
    )(x, w1cat)


def _combine_body(p_ref, cnt_ref, xr_ref, b1_ref, w2p_ref, r2_ref,
                  big_ref, hr_ref):
    cnt = cnt_ref[0, :, 0:1] + cnt_ref[1, :, 0:1]
    h = (p_ref[0] + p_ref[1]) / jnp.maximum(cnt, 1.0)
    h = h + xr_ref[...] + b1_ref[...]
    h = jnp.maximum(h, 0.0)
    for r in range(R):
        big_ref[r] = jnp.dot(h, w2p_ref[r], preferred_element_type=jnp.float32)
    hr_ref[...] = jnp.dot(h, r2_ref[...], preferred_element_type=jnp.float32)


def _combine(p1, cnt, xroot, b1, w2pad, root2):
    return pl.pallas_call(
        _combine_body,
        grid=(NBLK,),
        in_specs=[
            pl.BlockSpec((2, NB, H), lambda i: (0, i, 0)),
            pl.BlockSpec((2, NB, 8), lambda i: (0, i, 0)),
            pl.BlockSpec((NB, H), lambda i: (i, 0)),
            pl.BlockSpec((1, H), lambda i: (0, 0)),
            pl.BlockSpec((R, H, H), lambda i: (0, 0, 0)),
            pl.BlockSpec((H, C), lambda i: (0, 0)),
        ],
        out_specs=(pl.BlockSpec((R, NB, H), lambda i: (0, i, 0)),
                   pl.BlockSpec((NB, C), lambda i: (i, 0))),
        out_shape=(jax.ShapeDtypeStruct((R, N, H), jnp.float32),
                   jax.ShapeDtypeStruct((N, C), jnp.float32)),
    )(p1, cnt, xroot, b1, w2pad, root2)


def _final_body(p_ref, cnt_ref, hr_ref, b2_ref, out_ref):
    cnt = cnt_ref[0, :, 0:1] + cnt_ref[1, :, 0:1]
    s = p_ref[0] + p_ref[1]
    m16 = s[:, 0:C]
    for g in range(1, R):
        m16 = m16 + s[:, g * C:(g + 1) * C]
    z = m16 / jnp.maximum(cnt, 1.0)
    z = z + hr_ref[...] + b2_ref[...]
    m = jnp.max(z, axis=1, keepdims=True)
    e = jnp.exp(z - m)
    lse = jnp.log(jnp.sum(e, axis=1, keepdims=True))
    out_ref[...] = z - m - lse


def _final(p2, cnt, h2root, b2):
    return pl.pallas_call(
        _final_body,
        grid=(NBLK,),
        in_specs=[
            pl.BlockSpec((2, NB, H), lambda i: (0, i, 0)),
            pl.BlockSpec((2, NB, 8), lambda i: (0, i, 0)),
            pl.BlockSpec((NB, C), lambda i: (i, 0)),
            pl.BlockSpec((1, C), lambda i: (0, 0)),
        ],
        out_specs=pl.BlockSpec((NB, C), lambda i: (i, 0)),
        out_shape=jax.ShapeDtypeStruct((N, C), jnp.float32),
    )(p2, cnt, h2root, b2)


def _agg_body(table, fidx, dsts, zh, out_p,
              idx0, idx1, dst0, dst1, rows0, rows1, acc_sh,
              sg0, sg1, ss0, ss1):
    c = lax.axis_index("c")
    s = lax.axis_index("s")
    wid = c * 16 + s
    ebase = wid * EPW

    pltpu.sync_copy(zh.at[pl.ds(s * TROWS, TROWS)],
                    acc_sh.at[pl.ds(s * TROWS, TROWS)])
    plsc.subcore_barrier()

    pltpu.sync_copy(fidx.at[pl.ds(ebase, K)], idx0)
    pltpu.sync_copy(dsts.at[pl.ds(ebase, K)], dst0)
    pltpu.async_copy(table.at[idx0], rows0, sg0)

    def body(i, _):
        a = 2 * i
        pltpu.make_async_copy(table.at[idx0], rows0, sg0).wait()
        pltpu.async_copy(rows0, acc_sh.at[dst0], ss0, add=True)
        base1 = ebase + (a + 1) * K
        pltpu.sync_copy(fidx.at[pl.ds(base1, K)], idx1)
        pltpu.sync_copy(dsts.at[pl.ds(base1, K)], dst1)
        pltpu.async_copy(table.at[idx1], rows1, sg1)
        pltpu.make_async_copy(table.at[idx1], rows1, sg1).wait()
        pltpu.async_copy(rows1, acc_sh.at[dst1], ss1, add=True)
        pltpu.make_async_copy(rows0, acc_sh.at[dst0], ss0).wait()
        base2 = ebase + (a + 2) * K
        pltpu.sync_copy(fidx.at[pl.ds(base2, K)], idx0)
        pltpu.sync_copy(dsts.at[pl.ds(base2, K)], dst0)
        pltpu.async_copy(table.at[idx0], rows0, sg0)
        pltpu.make_async_copy(rows1, acc_sh.at[dst1], ss1).wait()
        return 0

    lax.fori_loop(0, (NCH - 1) // 2, body, 0)
    pltpu.make_async_copy(table.at[idx0], rows0, sg0).wait()
    pltpu.sync_copy(rows0, acc_sh.at[dst0], add=True)
    plsc.subcore_barrier()

    pltpu.sync_copy(acc_sh.at[pl.ds(s * TROWS, TROWS)],
                    out_p.at[pl.ds(c * NPAD + s * TROWS, TROWS)])


@functools.cache
def _get_agg():
    return pl.kernel(
        _agg_body,
        out_type=jax.ShapeDtypeStruct((2 * NPAD, H), jnp.float32),
        mesh=plsc.VectorSubcoreMesh(core_axis_name="c", subcore_axis_name="s"),
        scratch_types=[
            pltpu.VMEM((K,), jnp.int32),
            pltpu.VMEM((K,), jnp.int32),
            pltpu.VMEM((K,), jnp.int32),
            pltpu.VMEM((K,), jnp.int32),
            pltpu.VMEM((K, H), jnp.float32),
            pltpu.VMEM((K, H), jnp.float32),
            pltpu.VMEM_SHARED((NPAD, H), jnp.float32),
            pltpu.SemaphoreType.DMA,
            pltpu.SemaphoreType.DMA,
            pltpu.SemaphoreType.DMA,
            pltpu.SemaphoreType.DMA,
        ],
    )


def _cnt_body(dsts, zh, out_cnt, dst_v, ones_v, cnt_sh, sem):
    c = lax.axis_index("c")
    s = lax.axis_index("s")
    wid = c * 16 + s

    pltpu.sync_copy(zh.at[pl.ds(s * TROWS, TROWS)],
                    cnt_sh.at[pl.ds(s * TROWS, TROWS)])
    for j in range(K):
        for q in range(H // 16):
            ones_v[j, pl.ds(q * 16, 16)] = jnp.ones((16,), jnp.float32)
    plsc.subcore_barrier()

    def body(ci, _):
        base = wid * EPW + ci * K
        pltpu.sync_copy(dsts.at[pl.ds(base, K)], dst_v)
        pltpu.sync_copy(ones_v, cnt_sh.at[dst_v], add=True)
        return 0

    lax.fori_loop(0, NCH, body, 0)
    plsc.subcore_barrier()

    pltpu.sync_copy(cnt_sh.at[pl.ds(s * TROWS, TROWS)],
                    out_cnt.at[pl.ds(c * NPAD + s * TROWS, TROWS)])


@functools.cache
def _get_cnt():
    return pl.kernel(
        _cnt_body,
        out_type=jax.ShapeDtypeStruct((2 * NPAD, H), jnp.float32),
        mesh=plsc.VectorSubcoreMesh(core_axis_name="c", subcore_axis_name="s"),
        scratch_types=[
            pltpu.VMEM((K,), jnp.int32),
            pltpu.VMEM((K, H), jnp.float32),
            pltpu.VMEM_SHARED((NPAD, H), jnp.float32),
            pltpu.SemaphoreType.DMA,
        ],
    )


@jax.jit
def kernel(x, edge_index, edge_type, W1, root1, b1, W2, root2, b2):
    src = edge_index[0]
    dst = edge_index[1]
    fidx = edge_type * N + src

    w1cat = jnp.concatenate([W1, root1[None]], axis=0)
    eye = jnp.eye(R, dtype=jnp.float32)
    w2pad = jnp.einsum('rhc,rg->rhgc', W2, eye).reshape(R, H, R * C)

    zh = jnp.zeros((NPAD, H), jnp.float32)

    cntp = _get_cnt()(dst, zh)
    cnt = cntp.reshape(2, NPAD, H)[:, :, 0:8]

    h1t = _transform1(x, w1cat)
    p1 = _get_agg()(h1t.reshape(9 * N, H), fidx, dst, zh)
    big, h2root = _combine(p1.reshape(2, NPAD, H), cnt, h1t[8],
                           b1.reshape(1, H), w2pad, root2)
    p2 = _get_agg()(big.reshape(R * N, H), fidx, dst, zh)
    return _final(p2.reshape(2, NPAD, H), cnt, h2root, b2.reshape(1, C))


# speedup vs baseline: 21.2600x; 1.3917x over previous
import functools

import jax
import jax.numpy as jnp
from jax import lax
from jax.experimental import pallas as pl
from jax.experimental.pallas import tpu as pltpu
from jax.experimental.pallas import tpu_sc as plsc

N = 10000
E = 320000
R = 8
D = 128
H = 128
C = 16

NB = 400
NBLK = N // NB

NW = 32
EPW = E // NW
K = 80
NCH = EPW // K
NPAD = 10240
TROWS = NPAD // 16


def _mm9_body(x_ref, w_ref, out_ref):
    x = x_ref[...]
    for r in range(9):
        out_ref[r] = jnp.dot(x, w_ref[r], preferred_element_type=jnp.float32)


def _transform1(x, w1cat):
    return pl.pallas_call(
        _mm9_body,
        grid=(NBLK,),
        in_specs=[
            pl.BlockSpec((NB, D), lambda i: (i, 0)),
            pl.BlockSpec((9, D, H), lambda i: (0, 0, 0)),
        ],
        out_specs=pl.BlockSpec((9, NB, H), lambda i: (0, i, 0)),
        out_shape=jax.ShapeDtypeStruct((9, N, H), jnp.float32),
    )(x, w1cat)


def _combine_body(p_ref, cnt_ref, xr_ref, b1_ref, w2p_ref, r2_ref,
                  big_ref, hr_ref):
    cnt = cnt_ref[0, :, 0:1] + cnt_ref[1, :, 0:1]
    h = (p_ref[0] + p_ref[1]) / jnp.maximum(cnt, 1.0)
    h = h + xr_ref[...] + b1_ref[...]
    h = jnp.maximum(h, 0.0)
    for r in range(R):
        big_ref[r] = jnp.dot(h, w2p_ref[r], preferred_element_type=jnp.float32)
    hr_ref[...] = jnp.dot(h, r2_ref[...], preferred_element_type=jnp.float32)


def _combine(p1, cnt, xroot, b1, w2pad, root2):
    return pl.pallas_call(
        _combine_body,
        grid=(NBLK,),
        in_specs=[
            pl.BlockSpec((2, NB, H), lambda i: (0, i, 0)),
            pl.BlockSpec((2, NB, 8), lambda i: (0, i, 0)),
            pl.BlockSpec((NB, H), lambda i: (i, 0)),
            pl.BlockSpec((1, H), lambda i: (0, 0)),
            pl.BlockSpec((R, H, H), lambda i: (0, 0, 0)),
            pl.BlockSpec((H, C), lambda i: (0, 0)),
        ],
        out_specs=(pl.BlockSpec((R, NB, H), lambda i: (0, i, 0)),
                   pl.BlockSpec((NB, C), lambda i: (i, 0))),
        out_shape=(jax.ShapeDtypeStruct((R, N, H), jnp.float32),
                   jax.ShapeDtypeStruct((N, C), jnp.float32)),
    )(p1, cnt, xroot, b1, w2pad, root2)


def _final_body(p_ref, cnt_ref, hr_ref, b2_ref, out_ref):
    cnt = cnt_ref[0, :, 0:1] + cnt_ref[1, :, 0:1]
    s = p_ref[0] + p_ref[1]
    m16 = s[:, 0:C]
    for g in range(1, R):
        m16 = m16 + s[:, g * C:(g + 1) * C]
    z = m16 / jnp.maximum(cnt, 1.0)
    z = z + hr_ref[...] + b2_ref[...]
    m = jnp.max(z, axis=1, keepdims=True)
    e = jnp.exp(z - m)
    lse = jnp.log(jnp.sum(e, axis=1, keepdims=True))
    out_ref[...] = z - m - lse


def _final(p2, cnt, h2root, b2):
    return pl.pallas_call(
        _final_body,
        grid=(NBLK,),
        in_specs=[
            pl.BlockSpec((2, NB, H), lambda i: (0, i, 0)),
            pl.BlockSpec((2, NB, 8), lambda i: (0, i, 0)),
            pl.BlockSpec((NB, C), lambda i: (i, 0)),
            pl.BlockSpec((1, C), lambda i: (0, 0)),
        ],
        out_specs=pl.BlockSpec((NB, C), lambda i: (i, 0)),
        out_shape=jax.ShapeDtypeStruct((N, C), jnp.float32),
    )(p2, cnt, h2root, b2)


def _agg_body(table, fidx, dsts, zh, out_p,
              idx0, idx1, idx2, dst0, dst1, dst2, rows0, rows1, rows2, acc_sh,
              sg0, sg1, sg2, ss0, ss1, ss2):
    c = lax.axis_index("c")
    s = lax.axis_index("s")
    wid = c * 16 + s
    ebase = wid * EPW

    idx = (idx0, idx1, idx2)
    dst = (dst0, dst1, dst2)
    rows = (rows0, rows1, rows2)
    sg = (sg0, sg1, sg2)
    ss = (ss0, ss1, ss2)

    pltpu.sync_copy(zh.at[pl.ds(s * TROWS, TROWS)],
                    acc_sh.at[pl.ds(s * TROWS, TROWS)])
    plsc.subcore_barrier()

    def load_and_gather(b, base):
        pltpu.sync_copy(fidx.at[pl.ds(base, K)], idx[b])
        pltpu.sync_copy(dsts.at[pl.ds(base, K)], dst[b])
        pltpu.async_copy(table.at[idx[b]], rows[b], sg[b])

    def wait_gather(b):
        pltpu.make_async_copy(table.at[idx[b]], rows[b], sg[b]).wait()

    def start_scatter(b):
        pltpu.async_copy(rows[b], acc_sh.at[dst[b]], ss[b], add=True)

    def wait_scatter(b):
        pltpu.make_async_copy(rows[b], acc_sh.at[dst[b]], ss[b]).wait()

    load_and_gather(0, ebase)
    load_and_gather(1, ebase + K)
    wait_gather(0)
    start_scatter(0)
    load_and_gather(2, ebase + 2 * K)

    def body(i, _):
        t0 = 3 * i + 1
        for u in range(3):
            b = (1 + u) % 3
            wait_gather(b)
            start_scatter(b)
            wait_scatter((b + 2) % 3)
            load_and_gather((b + 2) % 3, ebase + (t0 + u + 2) * K)
        return 0

    lax.fori_loop(0, (NCH - 2) // 3, body, 0)
    wait_gather(1)
    start_scatter(1)
    wait_gather(2)
    wait_scatter(0)
    wait_scatter(1)
    plsc.subcore_barrier()

    pltpu.sync_copy(acc_sh.at[pl.ds(s * TROWS, TROWS)],
                    out_p.at[pl.ds(c * NPAD + s * TROWS, TROWS)])


@functools.cache
def _get_agg():
    return pl.kernel(
        _agg_body,
        out_type=jax.ShapeDtypeStruct((2 * NPAD, H), jnp.float32),
        mesh=plsc.VectorSubcoreMesh(core_axis_name="c", subcore_axis_name="s"),
        scratch_types=[
            pltpu.VMEM((K,), jnp.int32),
            pltpu.VMEM((K,), jnp.int32),
            pltpu.VMEM((K,), jnp.int32),
            pltpu.VMEM((K,), jnp.int32),
            pltpu.VMEM((K,), jnp.int32),
            pltpu.VMEM((K,), jnp.int32),
            pltpu.VMEM((K, H), jnp.float32),
            pltpu.VMEM((K, H), jnp.float32),
            pltpu.VMEM((K, H), jnp.float32),
            pltpu.VMEM_SHARED((NPAD, H), jnp.float32),
            pltpu.SemaphoreType.DMA,
            pltpu.SemaphoreType.DMA,
            pltpu.SemaphoreType.DMA,
            pltpu.SemaphoreType.DMA,
            pltpu.SemaphoreType.DMA,
            pltpu.SemaphoreType.DMA,
        ],
    )


def _cnt_body(dsts, zh, out_cnt, dst_v, ones_v, cnt_sh, sem):
    c = lax.axis_index("c")
    s = lax.axis_index("s")
    wid = c * 16 + s

    pltpu.sync_copy(zh.at[pl.ds(s * TROWS, TROWS)],
                    cnt_sh.at[pl.ds(s * TROWS, TROWS)])
    for j in range(K):
        for q in range(H // 16):
            ones_v[j, pl.ds(q * 16, 16)] = jnp.ones((16,), jnp.float32)
    plsc.subcore_barrier()

    def body(ci, _):
        base = wid * EPW + ci * K
        pltpu.sync_copy(dsts.at[pl.ds(base, K)], dst_v)
        pltpu.sync_copy(ones_v, cnt_sh.at[dst_v], add=True)
        return 0

    lax.fori_loop(0, NCH, body, 0)
    plsc.subcore_barrier()

    pltpu.sync_copy(cnt_sh.at[pl.ds(s * TROWS, TROWS)],
                    out_cnt.at[pl.ds(c * NPAD + s * TROWS, TROWS)])


@functools.cache
def _get_cnt():
    return pl.kernel(
        _cnt_body,
        out_type=jax.ShapeDtypeStruct((2 * NPAD, H), jnp.float32),
        mesh=plsc.VectorSubcoreMesh(core_axis_name="c", subcore_axis_name="s"),
        scratch_types=[
            pltpu.VMEM((K,), jnp.int32),
            pltpu.VMEM((K, H), jnp.float32),
            pltpu.VMEM_SHARED((NPAD, H), jnp.float32),
            pltpu.SemaphoreType.DMA,
        ],
    )


@jax.jit
def kernel(x, edge_index, edge_type, W1, root1, b1, W2, root2, b2):
    src = edge_index[0]
    dst = edge_index[1]
    fidx = edge_type * N + src

    w1cat = jnp.concatenate([W1, root1[None]], axis=0)
    eye = jnp.eye(R, dtype=jnp.float32)
    w2pad = jnp.einsum('rhc,rg->rhgc', W2, eye).reshape(R, H, R * C)

    zh = jnp.zeros((NPAD, H), jnp.float32)

    fidx = jnp.concatenate([fidx, jnp.zeros((K,), jnp.int32)])
    dstp = jnp.concatenate([dst, jnp.zeros((K,), jnp.int32)])

    cntp = _get_cnt()(dst, zh)
    cnt = cntp.reshape(2, NPAD, H)[:, :, 0:8]

    h1t = _transform1(x, w1cat)
    p1 = _get_agg()(h1t.reshape(9 * N, H), fidx, dstp, zh)
    big, h2root = _combine(p1.reshape(2, NPAD, H), cnt, h1t[8],
                           b1.reshape(1, H), w2pad, root2)
    p2 = _get_agg()(big.reshape(R * N, H), fidx, dstp, zh)
    return _final(p2.reshape(2, NPAD, H), cnt, h2root, b2.reshape(1, C))


# speedup vs baseline: 24.8803x; 1.1703x over previous
import functools

import jax
import jax.numpy as jnp
from jax import lax
from jax.experimental import pallas as pl
from jax.experimental.pallas import tpu as pltpu
from jax.experimental.pallas import tpu_sc as plsc

N = 10000
E = 320000
R = 8
D = 128
H = 128
C = 16

NB = 400
NBLK = N // NB

NW = 32
EPW = E // NW
K = 80
NCH = EPW // K
NPAD = 10240
TROWS = NPAD // 16


def _mm9_body(x_ref, w_ref, out_ref):
    x = x_ref[...]
    for r in range(9):
        out_ref[r] = jnp.dot(x, w_ref[r], preferred_element_type=jnp.float32)


def _transform1(x, w1cat):
    return pl.pallas_call(
        _mm9_body,
        grid=(NBLK,),
        in_specs=[
            pl.BlockSpec((NB, D), lambda i: (i, 0)),
            pl.BlockSpec((9, D, H), lambda i: (0, 0, 0)),
        ],
        out_specs=pl.BlockSpec((9, NB, H), lambda i: (0, i, 0)),
        out_shape=jax.ShapeDtypeStruct((9, N, H), jnp.float32),
    )(x, w1cat)


def _combine_body(p_ref, cnt_ref, xr_ref, b1_ref, w2p_ref, r2_ref,
                  big_ref, hr_ref):
    cnt = cnt_ref[0, :, 0:1] + cnt_ref[1, :, 0:1]
    h = (p_ref[0] + p_ref[1]) / jnp.maximum(cnt, 1.0)
    h = h + xr_ref[...] + b1_ref[...]
    h = jnp.maximum(h, 0.0)
    for r in range(R):
        big_ref[r] = jnp.dot(h, w2p_ref[r], preferred_element_type=jnp.float32)
    hr_ref[...] = jnp.dot(h, r2_ref[...], preferred_element_type=jnp.float32)


def _combine(p1, cnt, xroot, b1, w2pad, root2):
    return pl.pallas_call(
        _combine_body,
        grid=(NBLK,),
        in_specs=[
            pl.BlockSpec((2, NB, H), lambda i: (0, i, 0)),
            pl.BlockSpec((2, NB, 8), lambda i: (0, i, 0)),
            pl.BlockSpec((NB, H), lambda i: (i, 0)),
            pl.BlockSpec((1, H), lambda i: (0, 0)),
            pl.BlockSpec((R, H, H), lambda i: (0, 0, 0)),
            pl.BlockSpec((H, C), lambda i: (0, 0)),
        ],
        out_specs=(pl.BlockSpec((R, NB, H), lambda i: (0, i, 0)),
                   pl.BlockSpec((NB, C), lambda i: (i, 0))),
        out_shape=(jax.ShapeDtypeStruct((R, N, H), jnp.float32),
                   jax.ShapeDtypeStruct((N, C), jnp.float32)),
    )(p1, cnt, xroot, b1, w2pad, root2)


def _final_body(p_ref, cnt_ref, hr_ref, b2_ref, out_ref):
    cnt = cnt_ref[0, :, 0:1] + cnt_ref[1, :, 0:1]
    s = p_ref[0] + p_ref[1]
    m16 = s[:, 0:C]
    for g in range(1, R):
        m16 = m16 + s[:, g * C:(g + 1) * C]
    z = m16 / jnp.maximum(cnt, 1.0)
    z = z + hr_ref[...] + b2_ref[...]
    m = jnp.max(z, axis=1, keepdims=True)
    e = jnp.exp(z - m)
    lse = jnp.log(jnp.sum(e, axis=1, keepdims=True))
    out_ref[...] = z - m - lse


def _final(p2, cnt, h2root, b2):
    return pl.pallas_call(
        _final_body,
        grid=(NBLK,),
        in_specs=[
            pl.BlockSpec((2, NB, H), lambda i: (0, i, 0)),
            pl.BlockSpec((2, NB, 8), lambda i: (0, i, 0)),
            pl.BlockSpec((NB, C), lambda i: (i, 0)),
            pl.BlockSpec((1, C), lambda i: (0, 0)),
        ],
        out_specs=pl.BlockSpec((NB, C), lambda i: (i, 0)),
        out_shape=jax.ShapeDtypeStruct((N, C), jnp.float32),
    )(p2, cnt, h2root, b2)


def _agg_body(table, fidx, dsts, zh, out_p,
              idx0, idx1, idx2, idx3, dst0, dst1, dst2, dst3,
              rows0, rows1, rows2, rows3, acc_sh,
              sg0, sg1, sg2, sg3, ss0, ss1, ss2, ss3):
    c = lax.axis_index("c")
    s = lax.axis_index("s")
    wid = c * 16 + s
    ebase = wid * EPW

    idx = (idx0, idx1, idx2, idx3)
    dst = (dst0, dst1, dst2, dst3)
    rows = (rows0, rows1, rows2, rows3)
    sg = (sg0, sg1, sg2, sg3)
    ss = (ss0, ss1, ss2, ss3)

    pltpu.sync_copy(zh.at[pl.ds(s * TROWS, TROWS)],
                    acc_sh.at[pl.ds(s * TROWS, TROWS)])
    plsc.subcore_barrier()

    def load_and_gather(b, base):
        pltpu.sync_copy(fidx.at[pl.ds(base, K)], idx[b])
        pltpu.sync_copy(dsts.at[pl.ds(base, K)], dst[b])
        pltpu.async_copy(table.at[idx[b]], rows[b], sg[b])

    def wait_gather(b):
        pltpu.make_async_copy(table.at[idx[b]], rows[b], sg[b]).wait()

    def start_scatter(b):
        pltpu.async_copy(rows[b], acc_sh.at[dst[b]], ss[b], add=True)

    def wait_scatter(b):
        pltpu.make_async_copy(rows[b], acc_sh.at[dst[b]], ss[b]).wait()

    load_and_gather(0, ebase)
    load_and_gather(1, ebase + K)
    load_and_gather(2, ebase + 2 * K)
    wait_gather(0)
    start_scatter(0)
    load_and_gather(3, ebase + 3 * K)

    def body(i, _):
        t0 = 4 * i + 1
        for u in range(4):
            b = (1 + u) % 4
            wait_gather(b)
            start_scatter(b)
            wait_scatter((b + 3) % 4)
            load_and_gather((b + 3) % 4, ebase + (t0 + u + 3) * K)
        return 0

    lax.fori_loop(0, (NCH - 5) // 4, body, 0)
    wait_gather(1)
    start_scatter(1)
    wait_scatter(0)
    load_and_gather(0, ebase + (NCH - 1) * K)
    wait_gather(2)
    start_scatter(2)
    wait_gather(3)
    start_scatter(3)
    wait_gather(0)
    start_scatter(0)
    wait_scatter(1)
    wait_scatter(2)
    wait_scatter(3)
    wait_scatter(0)
    plsc.subcore_barrier()

    pltpu.sync_copy(acc_sh.at[pl.ds(s * TROWS, TROWS)],
                    out_p.at[pl.ds(c * NPAD + s * TROWS, TROWS)])


@functools.cache
def _get_agg():
    return pl.kernel(
        _agg_body,
        out_type=jax.ShapeDtypeStruct((2 * NPAD, H), jnp.float32),
        mesh=plsc.VectorSubcoreMesh(core_axis_name="c", subcore_axis_name="s"),
        scratch_types=(
            [pltpu.VMEM((K,), jnp.int32)] * 8
            + [pltpu.VMEM((K, H), jnp.float32)] * 4
            + [pltpu.VMEM_SHARED((NPAD, H), jnp.float32)]
            + [pltpu.SemaphoreType.DMA] * 8
        ),
    )


def _cnt_body(dsts, zh, out_cnt, dst0, dst1, ones_v, cnt_sh, ss0, ss1):
    c = lax.axis_index("c")
    s = lax.axis_index("s")
    wid = c * 16 + s
    ebase = wid * EPW

    pltpu.sync_copy(zh.at[pl.ds(s * TROWS, TROWS)],
                    cnt_sh.at[pl.ds(s * TROWS, TROWS)])
    for j in range(K):
        for q in range(H // 16):
            ones_v[j, pl.ds(q * 16, 16)] = jnp.ones((16,), jnp.float32)
    plsc.subcore_barrier()

    pltpu.sync_copy(dsts.at[pl.ds(ebase, K)], dst0)

    def body(i, _):
        a = 2 * i
        pltpu.async_copy(ones_v, cnt_sh.at[dst0], ss0, add=True)
        pltpu.sync_copy(dsts.at[pl.ds(ebase + (a + 1) * K, K)], dst1)
        pltpu.async_copy(ones_v, cnt_sh.at[dst1], ss1, add=True)
        pltpu.make_async_copy(ones_v, cnt_sh.at[dst0], ss0).wait()
        pltpu.sync_copy(dsts.at[pl.ds(ebase + (a + 2) * K, K)], dst0)
        pltpu.make_async_copy(ones_v, cnt_sh.at[dst1], ss1).wait()
        return 0

    lax.fori_loop(0, (NCH - 1) // 2, body, 0)
    pltpu.sync_copy(ones_v, cnt_sh.at[dst0], add=True)
    plsc.subcore_barrier()

    pltpu.sync_copy(cnt_sh.at[pl.ds(s * TROWS, TROWS)],
                    out_cnt.at[pl.ds(c * NPAD + s * TROWS, TROWS)])


@functools.cache
def _get_cnt():
    return pl.kernel(
        _cnt_body,
        out_type=jax.ShapeDtypeStruct((2 * NPAD, H), jnp.float32),
        mesh=plsc.VectorSubcoreMesh(core_axis_name="c", subcore_axis_name="s"),
        scratch_types=[
            pltpu.VMEM((K,), jnp.int32),
            pltpu.VMEM((K,), jnp.int32),
            pltpu.VMEM((K, H), jnp.float32),
            pltpu.VMEM_SHARED((NPAD, H), jnp.float32),
            pltpu.SemaphoreType.DMA,
            pltpu.SemaphoreType.DMA,
        ],
    )


@jax.jit
def kernel(x, edge_index, edge_type, W1, root1, b1, W2, root2, b2):
    src = edge_index[0]
    dst = edge_index[1]
    fidx = edge_type * N + src

    w1cat = jnp.concatenate([W1, root1[None]], axis=0)
    eye = jnp.eye(R, dtype=jnp.float32)
    w2pad = jnp.einsum('rhc,rg->rhgc', W2, eye).reshape(R, H, R * C)

    zh = jnp.zeros((NPAD, H), jnp.float32)

    fidx = jnp.concatenate([fidx, jnp.zeros((K,), jnp.int32)])
    dstp = jnp.concatenate([dst, jnp.zeros((K,), jnp.int32)])

    cntp = _get_cnt()(dstp, zh)
    cnt = cntp.reshape(2, NPAD, H)[:, :, 0:8]

    h1t = _transform1(x, w1cat)
    p1 = _get_agg()(h1t.reshape(9 * N, H), fidx, dstp, zh)
    big, h2root = _combine(p1.reshape(2, NPAD, H), cnt, h1t[8],
                           b1.reshape(1, H), w2pad, root2)
    p2 = _get_agg()(big.reshape(R * N, H), fidx, dstp, zh)
    return _final(p2.reshape(2, NPAD, H), cnt, h2root, b2.reshape(1, C))
